# baseline (speedup 1.0000x reference)
import functools

import jax
import jax.numpy as jnp
from jax import lax
from jax.experimental import pallas as pl
from jax.experimental.pallas import tpu as pltpu
from jax.experimental.pallas import tpu_sc as plsc

_B = 2
_C = 64
_D = 64
_N = 100000
_S = _B * _D * _D * _D
_SPB = _D * _D * _D
_NT = 16
_SLOTS_T = _SPB // _NT
_NPAD = 100352
_SENT = _N
_NP2 = 102400
_CH = 2048
_NCHUNK = _NP2 // _CH
_GRP = _CH // 16
_CS = 2048
_CPP = _SPB // _CS
_PLANES = 4
_NCK = _PLANES * _CPP
_TB = 1024


def _tc_transpose_body(x_ref, o_ref):
    i = pl.program_id(0)
    rows = jax.lax.broadcasted_iota(jnp.int32, (_TB, _C), 0) + i * _TB
    x = jnp.where(rows < _N, x_ref[...], 0.0)
    o_ref[...] = x.T


def _transpose_features(features):
    return pl.pallas_call(
        _tc_transpose_body,
        grid=(_NPAD // _TB,),
        in_specs=[pl.BlockSpec((_TB, _C), lambda i: (i, 0))],
        out_specs=pl.BlockSpec((_C, _TB), lambda i: (0, i)),
        out_shape=jax.ShapeDtypeStruct((_C, _NPAD), jnp.float32),
    )(features)


def _sc_body(dest, featT, out, m_out, mloc, mb0, mb1,
             ob0, ob1, row, semm0, semm1, semo0, semo1):
    sc = lax.axis_index("c")
    tid = lax.axis_index("s")
    mybase = sc * _SPB + tid * _SLOTS_T

    iota = lax.iota(jnp.int32, 16)
    sent = jnp.full((16,), _SENT, dtype=jnp.int32)

    def build_pass():
        lo = mybase

        def init(i, _):
            for u in range(8):
                mloc[pl.ds((i * 8 + u) * 16, 16)] = sent
            return 0

        lax.fori_loop(0, _SLOTS_T // 128, init, 0)

        def consume(t, dbuf):
            off = t * _CH

            def grp(i, _):
                ds = [dbuf[pl.ds((i * 4 + u) * 16, 16)] for u in range(4)]
                for u in range(4):
                    loc = ds[u] - lo
                    ok = (loc >= 0) & (loc < _SLOTS_T)
                    locc = loc & (_SLOTS_T - 1)
                    pidx = iota + (off + (i * 4 + u) * 16)
                    plsc.store_scatter(mloc, [locc], pidx, mask=ok)
                return 0

            lax.fori_loop(0, _GRP // 4, grp, 0)

        pltpu.async_copy(dest.at[pl.ds(0, _CH)], mb0.at[pl.ds(0, _CH)],
                         semm0)
        pltpu.async_copy(dest.at[pl.ds(_CH, _CH)], mb1.at[pl.ds(0, _CH)],
                         semm1)

        def pair(kk, _):
            t0 = kk * 2
            pltpu.make_async_copy(dest.at[pl.ds(0, _CH)],
                                  mb0.at[pl.ds(0, _CH)], semm0).wait()
            consume(t0, mb0)

            @pl.when(t0 + 2 < _NCHUNK)
            def _():
                pltpu.async_copy(dest.at[pl.ds((t0 + 2) * _CH, _CH)],
                                 mb0.at[pl.ds(0, _CH)], semm0)

            pltpu.make_async_copy(dest.at[pl.ds(0, _CH)],
                                  mb1.at[pl.ds(0, _CH)], semm1).wait()
            consume(t0 + 1, mb1)

            @pl.when(t0 + 3 < _NCHUNK)
            def _():
                pltpu.async_copy(dest.at[pl.ds((t0 + 3) * _CH, _CH)],
                                 mb1.at[pl.ds(0, _CH)], semm1)

            return 0

        lax.fori_loop(0, _NCHUNK // 2, pair, 0)
        pltpu.sync_copy(mloc, m_out.at[pl.ds(lo, _SLOTS_T)])

    with jax.named_scope("build"):
        build_pass()
        plsc.subcore_barrier()

    obs = (ob0, ob1)
    semos = (semo0, semo1)
    _NBC = _NCK // 2

    def m_src(j):
        q = j & (_CPP // 2 - 1)
        return m_out.at[pl.ds(sc * _SPB + q * 2 * _CS, 2 * _CS)]

    def plane_of(k):
        return lax.div(k, _CPP)

    def emit(k, mbuf, half, obuf, semo):
        @pl.when((k & (_CPP - 1)) == 0)
        def _():
            with jax.named_scope("row"):
                c = tid * _PLANES + plane_of(k)
                pltpu.sync_copy(featT.at[pl.ds(c * _NPAD, _NPAD)], row)

        with jax.named_scope("gat"):
            @plsc.parallel_loop(0, 32, step=1, unroll=16)
            def gather(r):
                for u4 in range(4):
                    idx = mbuf[pl.ds(half * 2048 + r * 64 + u4 * 16, 16)]
                    obuf[r, pl.ds(u4 * 16, 16)] = plsc.load_gather(row,
                                                                   [idx])

        c = tid * _PLANES + plane_of(k)
        q = k & (_CPP - 1)
        pltpu.async_copy(
            obuf, out.at[sc, c, q >> 1, pl.ds((q & 1) * 32, 32)], semo)

    def o_drain(obuf, semo):
        pltpu.make_async_copy(obuf, out.at[sc, 0, 0, pl.ds(0, 32)],
                              semo).wait()

    pltpu.async_copy(m_src(0), mb0, semm0)
    pltpu.async_copy(m_src(1), mb1, semm1)

    def chunk_oct(jj, _):
        k0 = jj * 4
        for half_pair, mbuf, semm in ((0, mb0, semm0), (1, mb1, semm1)):
            kb = k0 + half_pair * 2
            with jax.named_scope("m_wait"):
                pltpu.make_async_copy(m_src(0), mbuf, semm).wait()
            for h in range(2):
                k = kb + h
                if half_pair == 0 and h == 0:
                    @pl.when(jj > 0)
                    def _():
                        with jax.named_scope("odrain"):
                            o_drain(obs[h], semos[h])
                elif half_pair == 0 and h == 1:
                    @pl.when(jj > 0)
                    def _():
                        with jax.named_scope("odrain"):
                            o_drain(obs[h], semos[h])
                else:
                    with jax.named_scope("odrain"):
                        o_drain(obs[h], semos[h])
                emit(k, mbuf, h, obs[h], semos[h])

            j = jj * 2 + half_pair
            @pl.when(j + 2 < _NBC)
            def _():
                pltpu.async_copy(m_src(j + 2), mbuf, semm)

        return 0

    lax.fori_loop(0, _NCK // 4, chunk_oct, 0)
    o_drain(ob0, semo0)
    o_drain(ob1, semo1)


@functools.cache
def _build_sc():
    @functools.partial(
        pl.kernel,
        out_type=(
            jax.ShapeDtypeStruct((_B, _C, _D, _D, _D), jnp.float32),
            jax.ShapeDtypeStruct((_S,), jnp.int32),
        ),
        mesh=plsc.VectorSubcoreMesh(core_axis_name="c", subcore_axis_name="s"),
        compiler_params=pltpu.CompilerParams(needs_layout_passes=False,
                                             use_tc_tiling_on_sc=False),
        scratch_types=[
            pltpu.VMEM((_SLOTS_T,), jnp.int32),
            pltpu.VMEM((2 * _CS,), jnp.int32),
            pltpu.VMEM((2 * _CS,), jnp.int32),
            pltpu.VMEM((32, 64), jnp.float32),
            pltpu.VMEM((32, 64), jnp.float32),
            pltpu.VMEM((_NPAD,), jnp.float32),
            pltpu.SemaphoreType.DMA,
            pltpu.SemaphoreType.DMA,
            pltpu.SemaphoreType.DMA,
            pltpu.SemaphoreType.DMA,
        ],
    )
    def _sparse_to_dense(dest, featT, out, m_out, *scratch):
        _sc_body(dest, featT, out, m_out, *scratch)

    return _sparse_to_dense


def kernel(features, batch_idx, coords):
    dest = ((batch_idx * _D + coords[:, 0]) * _D + coords[:, 1]) * _D \
        + coords[:, 2]
    dest = jnp.concatenate(
        [dest, jnp.full((_NP2 - _N,), _S, dtype=jnp.int32)])
    featT = _transpose_features(features).reshape(-1)
    dense, _ = _build_sc()(dest.astype(jnp.int32), featT)
    return dense


# speedup vs baseline: 1.0556x; 1.0556x over previous
import functools

import jax
import jax.numpy as jnp
from jax import lax
from jax.experimental import pallas as pl
from jax.experimental.pallas import tpu as pltpu
from jax.experimental.pallas import tpu_sc as plsc

_B = 2
_C = 64
_D = 64
_N = 100000
_S = _B * _D * _D * _D
_SPB = _D * _D * _D
_NT = 16
_SLOTS_T = _SPB // _NT
_NPAD = 100352
_SENT = _N
_NP2 = 102400
_CH = 2048
_NCHUNK = _NP2 // _CH
_GRP = _CH // 16
_CS = 2048
_CPP = _SPB // _CS
_PLANES = 4
_NCK = _PLANES * _CPP
_TB = 1024


def _tc_transpose_body(x_ref, o_ref):
    i = pl.program_id(0)
    rows = jax.lax.broadcasted_iota(jnp.int32, (_TB, _C), 0) + i * _TB
    x = jnp.where(rows < _N, x_ref[...], 0.0)
    o_ref[...] = x.T


def _transpose_features(features):
    return pl.pallas_call(
        _tc_transpose_body,
        grid=(_NPAD // _TB,),
        in_specs=[pl.BlockSpec((_TB, _C), lambda i: (i, 0))],
        out_specs=pl.BlockSpec((_C, _TB), lambda i: (0, i)),
        out_shape=jax.ShapeDtypeStruct((_C, _NPAD), jnp.float32),
    )(features)


def _sc_body(dest, featT, out, m_out, mloc, mb0, mb1, mb2, mb3,
             ob0, ob1, row, semm0, semm1, semm2, semm3, semo0, semo1):
    sc = lax.axis_index("c")
    tid = lax.axis_index("s")
    mybase = sc * _SPB + tid * _SLOTS_T

    iota = lax.iota(jnp.int32, 16)
    sent = jnp.full((16,), _SENT, dtype=jnp.int32)

    def build_pass():
        lo = mybase

        def init(i, _):
            for u in range(8):
                mloc[pl.ds((i * 8 + u) * 16, 16)] = sent
            return 0

        lax.fori_loop(0, _SLOTS_T // 128, init, 0)

        def consume(t, dbuf):
            off = t * _CH

            def grp(i, _):
                ds = [dbuf[pl.ds((i * 4 + u) * 16, 16)] for u in range(4)]
                for u in range(4):
                    loc = ds[u] - lo
                    ok = (loc >= 0) & (loc < _SLOTS_T)
                    locc = loc & (_SLOTS_T - 1)
                    pidx = iota + (off + (i * 4 + u) * 16)
                    plsc.store_scatter(mloc, [locc], pidx, mask=ok)
                return 0

            lax.fori_loop(0, _GRP // 4, grp, 0)

        pltpu.async_copy(dest.at[pl.ds(0, _CH)], mb0, semm0)
        pltpu.async_copy(dest.at[pl.ds(_CH, _CH)], mb1, semm1)

        def pair(kk, _):
            t0 = kk * 2
            pltpu.make_async_copy(dest.at[pl.ds(0, _CH)], mb0, semm0).wait()
            consume(t0, mb0)

            @pl.when(t0 + 2 < _NCHUNK)
            def _():
                pltpu.async_copy(dest.at[pl.ds((t0 + 2) * _CH, _CH)], mb0,
                                 semm0)

            pltpu.make_async_copy(dest.at[pl.ds(0, _CH)], mb1, semm1).wait()
            consume(t0 + 1, mb1)

            @pl.when(t0 + 3 < _NCHUNK)
            def _():
                pltpu.async_copy(dest.at[pl.ds((t0 + 3) * _CH, _CH)], mb1,
                                 semm1)

            return 0

        lax.fori_loop(0, _NCHUNK // 2, pair, 0)
        pltpu.sync_copy(mloc, m_out.at[pl.ds(lo, _SLOTS_T)])

    with jax.named_scope("build"):
        build_pass()
        plsc.subcore_barrier()

    mbs = (mb0, mb1, mb2, mb3)
    semms = (semm0, semm1, semm2, semm3)
    obs = (ob0, ob1)
    semos = (semo0, semo1)

    def m_src(k):
        q = k & (_CPP - 1)
        return m_out.at[pl.ds(sc * _SPB + q * _CS, _CS)]

    def plane_of(k):
        return lax.div(k, _CPP)

    def emit(k, mbuf, obuf, semo):
        @pl.when((k & (_CPP - 1)) == 0)
        def _():
            with jax.named_scope("row"):
                c = tid * _PLANES + plane_of(k)
                pltpu.sync_copy(featT.at[pl.ds(c * _NPAD, _NPAD)], row)

        with jax.named_scope("gat"):
            @plsc.parallel_loop(0, 32, step=1, unroll=16)
            def gather(r):
                for u4 in range(4):
                    idx = mbuf[pl.ds(r * 64 + u4 * 16, 16)]
                    obuf[r, pl.ds(u4 * 16, 16)] = plsc.load_gather(row,
                                                                   [idx])

        c = tid * _PLANES + plane_of(k)
        q = k & (_CPP - 1)
        pltpu.async_copy(
            obuf, out.at[sc, c, q >> 1, pl.ds((q & 1) * 32, 32)], semo)

    def o_drain(obuf, semo):
        pltpu.make_async_copy(obuf, out.at[sc, 0, 0, pl.ds(0, 32)],
                              semo).wait()

    for u in range(4):
        pltpu.async_copy(m_src(u), mbs[u], semms[u])

    def chunk_quad(kk, _):
        k0 = kk * 4
        for u in range(4):
            k = k0 + u
            with jax.named_scope("m_wait"):
                pltpu.make_async_copy(m_src(0), mbs[u], semms[u]).wait()

            if u < 2:
                @pl.when(kk > 0)
                def _():
                    with jax.named_scope("odrain"):
                        o_drain(obs[u & 1], semos[u & 1])
            else:
                with jax.named_scope("odrain"):
                    o_drain(obs[u & 1], semos[u & 1])

            emit(k, mbs[u], obs[u & 1], semos[u & 1])

            @pl.when(k + 4 < _NCK)
            def _():
                pltpu.async_copy(m_src(k + 4), mbs[u], semms[u])

        return 0

    lax.fori_loop(0, _NCK // 4, chunk_quad, 0)
    o_drain(ob0, semo0)
    o_drain(ob1, semo1)


@functools.cache
def _build_sc():
    @functools.partial(
        pl.kernel,
        out_type=(
            jax.ShapeDtypeStruct((_B, _C, _D, _D, _D), jnp.float32),
            jax.ShapeDtypeStruct((_S,), jnp.int32),
        ),
        mesh=plsc.VectorSubcoreMesh(core_axis_name="c", subcore_axis_name="s"),
        compiler_params=pltpu.CompilerParams(needs_layout_passes=False,
                                             use_tc_tiling_on_sc=False),
        scratch_types=[
            pltpu.VMEM((_SLOTS_T,), jnp.int32),
            pltpu.VMEM((_CS,), jnp.int32),
            pltpu.VMEM((_CS,), jnp.int32),
            pltpu.VMEM((_CS,), jnp.int32),
            pltpu.VMEM((_CS,), jnp.int32),
            pltpu.VMEM((32, 64), jnp.float32),
            pltpu.VMEM((32, 64), jnp.float32),
            pltpu.VMEM((_NPAD,), jnp.float32),
            pltpu.SemaphoreType.DMA,
            pltpu.SemaphoreType.DMA,
            pltpu.SemaphoreType.DMA,
            pltpu.SemaphoreType.DMA,
            pltpu.SemaphoreType.DMA,
            pltpu.SemaphoreType.DMA,
        ],
    )
    def _sparse_to_dense(dest, featT, out, m_out, *scratch):
        _sc_body(dest, featT, out, m_out, *scratch)

    return _sparse_to_dense


def kernel(features, batch_idx, coords):
    dest = ((batch_idx * _D + coords[:, 0]) * _D + coords[:, 1]) * _D \
        + coords[:, 2]
    dest = jnp.concatenate(
        [dest, jnp.full((_NP2 - _N,), _S, dtype=jnp.int32)])
    featT = _transpose_features(features).reshape(-1)
    dense, _ = _build_sc()(dest.astype(jnp.int32), featT)
    return dense
